# baseline (speedup 1.0000x reference)
import jax
import jax.numpy as jnp
from jax import lax
from jax.experimental import pallas as pl
from jax.experimental.pallas import tpu as pltpu
from jax.experimental.pallas import tpu_sc as plsc

_NUM_EXPERTS = 16
_CAPACITY = 1280
_TOKENS = 16384
_EMBED = 2048
_L = 16
_NVEC = _TOKENS // _L
_HALF = _CAPACITY // 2
_C = 16
_CZ = 8
_NCORES = 2
_NSUB = 16
_U = 4
_IDXCAP = _CAPACITY + 2 * _L
_TRASH = _IDXCAP + 2 * _L
_IDXLEN = _TRASH + _L


def _dispatch_kernel(x_hbm, ids_hbm, w_hbm,
                     xout_hbm, cw_hbm, ti_hbm, drop_hbm,
                     ids_v, idx_v, w_v, wout_v, tout_v, rows_v, rows2_v, zero_v,
                     tmp_v, red_v, shared_counts, sem, sem2, zsem):
    c = lax.axis_index("c")
    s = lax.axis_index("s")
    e = s
    lo = c * _HALF

    pltpu.sync_copy(ids_hbm, ids_v)
    pltpu.sync_copy(w_hbm, w_v)

    iota = lax.iota(jnp.int32, _L)

    def scan_body(i, cnt):
        for u in range(_U):
            v = i * _U + u
            ids16 = ids_v[pl.ds(v * _L, _L)]
            mi = jnp.where(ids16 == e, 1, 0)
            inc = plsc.cumsum(mi)
            rank = cnt + inc - mi
            keep = jnp.logical_and(mi > 0, rank < _IDXCAP)
            dest = jnp.where(keep, rank, _TRASH + iota)
            plsc.store_scatter(idx_v, [dest], v * _L + iota)
            cnt = cnt + inc[_L - 1]
        return cnt

    cnt = lax.fori_loop(0, _NVEC // _U, scan_body, jnp.int32(0))
    kept = jnp.minimum(cnt, _CAPACITY)

    idx_v[pl.ds(kept, _L)] = jnp.zeros((_L,), jnp.int32)
    idx_v[pl.ds(kept + _L, _L)] = jnp.zeros((_L,), jnp.int32)

    m_bal = ((11 * kept + 5 * _CAPACITY) // 32) // _C * _C
    m = jnp.minimum(m_bal, kept)
    lo_x = jnp.where(c == 0, 0, m)
    hi_x = jnp.where(c == 0, m, _CAPACITY)
    copy_n = jnp.clip(kept - lo_x, 0, hi_x - lo_x)

    for q in range(_CZ):
        for t in range(_EMBED // _L):
            zero_v[q, pl.ds(t * _L, _L)] = jnp.zeros((_L,), jnp.float32)

    zbase = jnp.where(c == 0, _CAPACITY, (kept + _CZ - 1) // _CZ * _CZ)
    nz = (_CAPACITY - zbase) // _CZ

    def zero_body(k, _):
        pltpu.async_copy(
            zero_v,
            xout_hbm.at[e, pl.ds(pl.multiple_of(zbase + k * _CZ, _CZ), _CZ)],
            zsem)
        return 0

    lax.fori_loop(0, nz, zero_body, 0)

    nfull = copy_n // _C
    rem = copy_n - nfull * _C

    def gather_start(k, buf, gsem):
        base = pl.multiple_of(lo_x + k * _C, _C)
        pltpu.async_copy(x_hbm.at[idx_v.at[pl.ds(base, _C)]], buf, gsem)

    def gather_wait(k, buf, gsem):
        base = pl.multiple_of(lo_x + k * _C, _C)
        pltpu.make_async_copy(x_hbm.at[idx_v.at[pl.ds(base, _C)]], buf,
                              gsem).wait()

    @pl.when(nfull > 0)
    def _():
        gather_start(0, rows_v, sem)

    for j in range(_HALF // _L):
        p = lo + j * _L + iota
        mk = p < kept
        idx16 = idx_v[pl.ds(lo + j * _L, _L)]
        idx_safe = jnp.where(mk, idx16, 0)
        w16 = plsc.load_gather(w_v, [idx_safe])
        wout_v[pl.ds(j * _L, _L)] = jnp.where(mk, w16, 0.0)
        tout_v[pl.ds(j * _L, _L)] = jnp.where(mk, idx16, -1)
    pltpu.sync_copy(wout_v, cw_hbm.at[e, pl.ds(lo, _HALF)])
    pltpu.sync_copy(tout_v, ti_hbm.at[e, pl.ds(lo, _HALF)])

    tmp_v[...] = jnp.full((_L,), cnt, jnp.int32)
    pltpu.sync_copy(tmp_v, shared_counts.at[s])
    plsc.subcore_barrier()

    @pl.when(s == 0)
    def _():
        pltpu.sync_copy(shared_counts, red_v)
        acc = jnp.zeros((_L,), jnp.int32)
        for ee in range(_NUM_EXPERTS):
            acc = acc + jnp.maximum(red_v[ee, :] - _CAPACITY, 0)
        tmp_v[...] = acc
        pltpu.sync_copy(tmp_v, drop_hbm)

    def copy_body(k, _):
        def step(buf, gsem, obuf, osem):
            gather_wait(k, buf, gsem)

            @pl.when(k + 1 < nfull)
            def _():
                gather_start(k + 1, obuf, osem)

            pltpu.sync_copy(
                buf, xout_hbm.at[e, pl.ds(pl.multiple_of(lo_x + k * _C, _C), _C)])

        @pl.when(k % 2 == 0)
        def _():
            step(rows_v, sem, rows2_v, sem2)

        @pl.when(k % 2 == 1)
        def _():
            step(rows2_v, sem2, rows_v, sem)

        return 0

    lax.fori_loop(0, nfull, copy_body, 0)

    @pl.when(rem > 0)
    def _():
        start = pl.multiple_of(lo_x + nfull * _C, _C)
        pltpu.async_copy(x_hbm.at[idx_v.at[pl.ds(start, _C)]], rows_v, sem).wait()
        for q in range(_C):
            @pl.when(q >= rem)
            def _():
                for t in range(_EMBED // _L):
                    rows_v[q, pl.ds(t * _L, _L)] = jnp.zeros((_L,), jnp.float32)
        pltpu.sync_copy(rows_v, xout_hbm.at[e, pl.ds(start, _C)])

    def zero_drain(k, _):
        pltpu.make_async_copy(
            zero_v,
            xout_hbm.at[e, pl.ds(pl.multiple_of(zbase + k * _CZ, _CZ), _CZ)],
            zsem).wait()
        return 0

    lax.fori_loop(0, nz, zero_drain, 0)


@jax.jit
def _dispatch(x, expert_ids, expert_weights):
    mesh = plsc.VectorSubcoreMesh(core_axis_name="c", subcore_axis_name="s",
                                  num_cores=_NCORES, num_subcores=_NSUB)
    kern = pl.kernel(
        _dispatch_kernel,
        out_type=(
            jax.ShapeDtypeStruct((_NUM_EXPERTS, _CAPACITY, _EMBED), jnp.float32),
            jax.ShapeDtypeStruct((_NUM_EXPERTS, _CAPACITY), jnp.float32),
            jax.ShapeDtypeStruct((_NUM_EXPERTS, _CAPACITY), jnp.int32),
            jax.ShapeDtypeStruct((_L,), jnp.int32),
        ),
        mesh=mesh,
        compiler_params=pltpu.CompilerParams(needs_layout_passes=False),
        scratch_types=(
            pltpu.VMEM((_TOKENS,), jnp.int32),
            pltpu.VMEM((_IDXLEN,), jnp.int32),
            pltpu.VMEM((_TOKENS,), jnp.float32),
            pltpu.VMEM((_HALF,), jnp.float32),
            pltpu.VMEM((_HALF,), jnp.int32),
            pltpu.VMEM((_C, _EMBED), jnp.float32),
            pltpu.VMEM((_C, _EMBED), jnp.float32),
            pltpu.VMEM((_CZ, _EMBED), jnp.float32),
            pltpu.VMEM((_L,), jnp.int32),
            pltpu.VMEM((_NSUB, _L), jnp.int32),
            pltpu.VMEM_SHARED((_NSUB, _L), jnp.int32),
            pltpu.SemaphoreType.DMA,
            pltpu.SemaphoreType.DMA,
            pltpu.SemaphoreType.DMA,
        ),
    )
    xo, cw, ti, drop = kern(x, expert_ids, expert_weights)
    return xo, cw, ti, drop[0]


def kernel(x, expert_ids, expert_weights):
    return _dispatch(x, expert_ids, expert_weights)


# speedup vs baseline: 1.0078x; 1.0078x over previous
import jax
import jax.numpy as jnp
from jax import lax
from jax.experimental import pallas as pl
from jax.experimental.pallas import tpu as pltpu
from jax.experimental.pallas import tpu_sc as plsc

_NUM_EXPERTS = 16
_CAPACITY = 1280
_TOKENS = 16384
_EMBED = 2048
_L = 16
_NVEC = _TOKENS // _L
_HALF = _CAPACITY // 2
_C = 16
_CZ = 8
_NCORES = 2
_NSUB = 16
_U = 4
_IDXCAP = _CAPACITY + 2 * _L
_TRASH = _IDXCAP + 2 * _L
_IDXLEN = _TRASH + _L


def _dispatch_kernel(x_hbm, ids_hbm, w_hbm,
                     xout_hbm, cw_hbm, ti_hbm, drop_hbm,
                     ids_v, idx_v, w_v, wout_v, tout_v, rows_v, rows2_v, zero_v,
                     tmp_v, red_v, shared_counts, sem, sem2, zsem):
    c = lax.axis_index("c")
    s = lax.axis_index("s")
    e = s
    lo = c * _HALF

    pltpu.sync_copy(ids_hbm, ids_v)
    pltpu.sync_copy(w_hbm, w_v)

    iota = lax.iota(jnp.int32, _L)

    def scan_body(i, cnt):
        for u in range(_U):
            v = i * _U + u
            ids16 = ids_v[pl.ds(v * _L, _L)]
            mi = jnp.where(ids16 == e, 1, 0)
            inc = plsc.cumsum(mi)
            rank = cnt + inc - mi
            keep = jnp.logical_and(mi > 0, rank < _IDXCAP)
            dest = jnp.where(keep, rank, _TRASH + iota)
            plsc.store_scatter(idx_v, [dest], v * _L + iota)
            cnt = cnt + inc[_L - 1]
        return cnt

    cnt = lax.fori_loop(0, _NVEC // _U, scan_body, jnp.int32(0))
    kept = jnp.minimum(cnt, _CAPACITY)

    idx_v[pl.ds(kept, _L)] = jnp.zeros((_L,), jnp.int32)
    idx_v[pl.ds(kept + _L, _L)] = jnp.zeros((_L,), jnp.int32)

    m_bal = ((7 * kept + 5 * _CAPACITY) // 24) // _C * _C
    m = jnp.minimum(m_bal, kept)
    lo_x = jnp.where(c == 0, 0, m)
    hi_x = jnp.where(c == 0, m, _CAPACITY)
    copy_n = jnp.clip(kept - lo_x, 0, hi_x - lo_x)

    for q in range(_CZ):
        for t in range(_EMBED // _L):
            zero_v[q, pl.ds(t * _L, _L)] = jnp.zeros((_L,), jnp.float32)

    zbase = jnp.where(c == 0, _CAPACITY, (kept + _CZ - 1) // _CZ * _CZ)
    nz = (_CAPACITY - zbase) // _CZ

    def zero_body(k, _):
        pltpu.async_copy(
            zero_v,
            xout_hbm.at[e, pl.ds(pl.multiple_of(zbase + k * _CZ, _CZ), _CZ)],
            zsem)
        return 0

    lax.fori_loop(0, nz, zero_body, 0)

    nfull = copy_n // _C
    rem = copy_n - nfull * _C

    def gather_start(k, buf, gsem):
        base = pl.multiple_of(lo_x + k * _C, _C)
        pltpu.async_copy(x_hbm.at[idx_v.at[pl.ds(base, _C)]], buf, gsem)

    def gather_wait(k, buf, gsem):
        base = pl.multiple_of(lo_x + k * _C, _C)
        pltpu.make_async_copy(x_hbm.at[idx_v.at[pl.ds(base, _C)]], buf,
                              gsem).wait()

    @pl.when(nfull > 0)
    def _():
        gather_start(0, rows_v, sem)

    for j in range(_HALF // _L):
        p = lo + j * _L + iota
        mk = p < kept
        idx16 = idx_v[pl.ds(lo + j * _L, _L)]
        idx_safe = jnp.where(mk, idx16, 0)
        w16 = plsc.load_gather(w_v, [idx_safe])
        wout_v[pl.ds(j * _L, _L)] = jnp.where(mk, w16, 0.0)
        tout_v[pl.ds(j * _L, _L)] = jnp.where(mk, idx16, -1)
    pltpu.sync_copy(wout_v, cw_hbm.at[e, pl.ds(lo, _HALF)])
    pltpu.sync_copy(tout_v, ti_hbm.at[e, pl.ds(lo, _HALF)])

    tmp_v[...] = jnp.full((_L,), cnt, jnp.int32)
    pltpu.sync_copy(tmp_v, shared_counts.at[s])
    plsc.subcore_barrier()

    @pl.when(s == 0)
    def _():
        pltpu.sync_copy(shared_counts, red_v)
        acc = jnp.zeros((_L,), jnp.int32)
        for ee in range(_NUM_EXPERTS):
            acc = acc + jnp.maximum(red_v[ee, :] - _CAPACITY, 0)
        tmp_v[...] = acc
        pltpu.sync_copy(tmp_v, drop_hbm)

    def copy_body(k, _):
        def step(buf, gsem, obuf, osem):
            gather_wait(k, buf, gsem)

            @pl.when(k + 1 < nfull)
            def _():
                gather_start(k + 1, obuf, osem)

            pltpu.sync_copy(
                buf, xout_hbm.at[e, pl.ds(pl.multiple_of(lo_x + k * _C, _C), _C)])

        @pl.when(k % 2 == 0)
        def _():
            step(rows_v, sem, rows2_v, sem2)

        @pl.when(k % 2 == 1)
        def _():
            step(rows2_v, sem2, rows_v, sem)

        return 0

    lax.fori_loop(0, nfull, copy_body, 0)

    @pl.when(rem > 0)
    def _():
        start = pl.multiple_of(lo_x + nfull * _C, _C)
        pltpu.async_copy(x_hbm.at[idx_v.at[pl.ds(start, _C)]], rows_v, sem).wait()
        for q in range(_C):
            @pl.when(q >= rem)
            def _():
                for t in range(_EMBED // _L):
                    rows_v[q, pl.ds(t * _L, _L)] = jnp.zeros((_L,), jnp.float32)
        pltpu.sync_copy(rows_v, xout_hbm.at[e, pl.ds(start, _C)])

    def zero_drain(k, _):
        pltpu.make_async_copy(
            zero_v,
            xout_hbm.at[e, pl.ds(pl.multiple_of(zbase + k * _CZ, _CZ), _CZ)],
            zsem).wait()
        return 0

    lax.fori_loop(0, nz, zero_drain, 0)


@jax.jit
def _dispatch(x, expert_ids, expert_weights):
    mesh = plsc.VectorSubcoreMesh(core_axis_name="c", subcore_axis_name="s",
                                  num_cores=_NCORES, num_subcores=_NSUB)
    kern = pl.kernel(
        _dispatch_kernel,
        out_type=(
            jax.ShapeDtypeStruct((_NUM_EXPERTS, _CAPACITY, _EMBED), jnp.float32),
            jax.ShapeDtypeStruct((_NUM_EXPERTS, _CAPACITY), jnp.float32),
            jax.ShapeDtypeStruct((_NUM_EXPERTS, _CAPACITY), jnp.int32),
            jax.ShapeDtypeStruct((_L,), jnp.int32),
        ),
        mesh=mesh,
        compiler_params=pltpu.CompilerParams(needs_layout_passes=False),
        scratch_types=(
            pltpu.VMEM((_TOKENS,), jnp.int32),
            pltpu.VMEM((_IDXLEN,), jnp.int32),
            pltpu.VMEM((_TOKENS,), jnp.float32),
            pltpu.VMEM((_HALF,), jnp.float32),
            pltpu.VMEM((_HALF,), jnp.int32),
            pltpu.VMEM((_C, _EMBED), jnp.float32),
            pltpu.VMEM((_C, _EMBED), jnp.float32),
            pltpu.VMEM((_CZ, _EMBED), jnp.float32),
            pltpu.VMEM((_L,), jnp.int32),
            pltpu.VMEM((_NSUB, _L), jnp.int32),
            pltpu.VMEM_SHARED((_NSUB, _L), jnp.int32),
            pltpu.SemaphoreType.DMA,
            pltpu.SemaphoreType.DMA,
            pltpu.SemaphoreType.DMA,
        ),
    )
    xo, cw, ti, drop = kern(x, expert_ids, expert_weights)
    return xo, cw, ti, drop[0]


def kernel(x, expert_ids, expert_weights):
    return _dispatch(x, expert_ids, expert_weights)


# speedup vs baseline: 1.0143x; 1.0064x over previous
import jax
import jax.numpy as jnp
from jax import lax
from jax.experimental import pallas as pl
from jax.experimental.pallas import tpu as pltpu
from jax.experimental.pallas import tpu_sc as plsc

_NUM_EXPERTS = 16
_CAPACITY = 1280
_TOKENS = 16384
_EMBED = 2048
_L = 16
_NVEC = _TOKENS // _L
_HALF = _CAPACITY // 2
_C = 16
_CZ = 8
_NCORES = 2
_NSUB = 16
_U = 4
_IDXCAP = _CAPACITY + 2 * _L
_TRASH = _IDXCAP + 2 * _L
_IDXLEN = _TRASH + _L


def _dispatch_kernel(x_hbm, ids_hbm, w_hbm,
                     xout_hbm, cw_hbm, ti_hbm, drop_hbm,
                     ids_v, idx_v, w_v, wout_v, tout_v, rows_v, rows2_v, zero_v,
                     tmp_v, red_v, shared_counts, sem, sem2, zsem):
    c = lax.axis_index("c")
    s = lax.axis_index("s")
    e = s
    lo = c * _HALF

    pltpu.sync_copy(ids_hbm, ids_v)
    pltpu.sync_copy(w_hbm, w_v)

    iota = lax.iota(jnp.int32, _L)

    def scan_body(i, cnt):
        for u in range(_U):
            v = i * _U + u
            ids16 = ids_v[pl.ds(v * _L, _L)]
            mi = jnp.where(ids16 == e, 1, 0)
            inc = plsc.cumsum(mi)
            rank = cnt + inc - mi
            keep = jnp.logical_and(mi > 0, rank < _IDXCAP)
            dest = jnp.where(keep, rank, _TRASH + iota)
            plsc.store_scatter(idx_v, [dest], v * _L + iota)
            cnt = cnt + inc[_L - 1]
        return cnt

    cnt = lax.fori_loop(0, _NVEC // _U, scan_body, jnp.int32(0))
    kept = jnp.minimum(cnt, _CAPACITY)

    idx_v[pl.ds(kept, _L)] = jnp.zeros((_L,), jnp.int32)
    idx_v[pl.ds(kept + _L, _L)] = jnp.zeros((_L,), jnp.int32)

    m_bal = ((kept + _CAPACITY) // 4) // _C * _C
    m = jnp.minimum(m_bal, kept)
    lo_x = jnp.where(c == 0, 0, m)
    hi_x = jnp.where(c == 0, m, _CAPACITY)
    copy_n = jnp.clip(kept - lo_x, 0, hi_x - lo_x)

    for q in range(_CZ):
        for t in range(_EMBED // _L):
            zero_v[q, pl.ds(t * _L, _L)] = jnp.zeros((_L,), jnp.float32)

    zbase = jnp.where(c == 0, _CAPACITY, (kept + _CZ - 1) // _CZ * _CZ)
    nz = (_CAPACITY - zbase) // _CZ

    def zero_body(k, _):
        pltpu.async_copy(
            zero_v,
            xout_hbm.at[e, pl.ds(pl.multiple_of(zbase + k * _CZ, _CZ), _CZ)],
            zsem)
        return 0

    lax.fori_loop(0, nz, zero_body, 0)

    nfull = copy_n // _C
    rem = copy_n - nfull * _C

    def gather_start(k, buf, gsem):
        base = pl.multiple_of(lo_x + k * _C, _C)
        pltpu.async_copy(x_hbm.at[idx_v.at[pl.ds(base, _C)]], buf, gsem)

    def gather_wait(k, buf, gsem):
        base = pl.multiple_of(lo_x + k * _C, _C)
        pltpu.make_async_copy(x_hbm.at[idx_v.at[pl.ds(base, _C)]], buf,
                              gsem).wait()

    @pl.when(nfull > 0)
    def _():
        gather_start(0, rows_v, sem)

    for j in range(_HALF // _L):
        p = lo + j * _L + iota
        mk = p < kept
        idx16 = idx_v[pl.ds(lo + j * _L, _L)]
        idx_safe = jnp.where(mk, idx16, 0)
        w16 = plsc.load_gather(w_v, [idx_safe])
        wout_v[pl.ds(j * _L, _L)] = jnp.where(mk, w16, 0.0)
        tout_v[pl.ds(j * _L, _L)] = jnp.where(mk, idx16, -1)
    pltpu.sync_copy(wout_v, cw_hbm.at[e, pl.ds(lo, _HALF)])
    pltpu.sync_copy(tout_v, ti_hbm.at[e, pl.ds(lo, _HALF)])

    tmp_v[...] = jnp.full((_L,), cnt, jnp.int32)
    pltpu.sync_copy(tmp_v, shared_counts.at[s])
    plsc.subcore_barrier()

    @pl.when(s == 0)
    def _():
        pltpu.sync_copy(shared_counts, red_v)
        acc = jnp.zeros((_L,), jnp.int32)
        for ee in range(_NUM_EXPERTS):
            acc = acc + jnp.maximum(red_v[ee, :] - _CAPACITY, 0)
        tmp_v[...] = acc
        pltpu.sync_copy(tmp_v, drop_hbm)

    def copy_body(k, _):
        def step(buf, gsem, obuf, osem):
            gather_wait(k, buf, gsem)

            @pl.when(k + 1 < nfull)
            def _():
                gather_start(k + 1, obuf, osem)

            pltpu.sync_copy(
                buf, xout_hbm.at[e, pl.ds(pl.multiple_of(lo_x + k * _C, _C), _C)])

        @pl.when(k % 2 == 0)
        def _():
            step(rows_v, sem, rows2_v, sem2)

        @pl.when(k % 2 == 1)
        def _():
            step(rows2_v, sem2, rows_v, sem)

        return 0

    lax.fori_loop(0, nfull, copy_body, 0)

    @pl.when(rem > 0)
    def _():
        start = pl.multiple_of(lo_x + nfull * _C, _C)
        pltpu.async_copy(x_hbm.at[idx_v.at[pl.ds(start, _C)]], rows_v, sem).wait()
        for q in range(_C):
            @pl.when(q >= rem)
            def _():
                for t in range(_EMBED // _L):
                    rows_v[q, pl.ds(t * _L, _L)] = jnp.zeros((_L,), jnp.float32)
        pltpu.sync_copy(rows_v, xout_hbm.at[e, pl.ds(start, _C)])

    def zero_drain(k, _):
        pltpu.make_async_copy(
            zero_v,
            xout_hbm.at[e, pl.ds(pl.multiple_of(zbase + k * _CZ, _CZ), _CZ)],
            zsem).wait()
        return 0

    lax.fori_loop(0, nz, zero_drain, 0)


@jax.jit
def _dispatch(x, expert_ids, expert_weights):
    mesh = plsc.VectorSubcoreMesh(core_axis_name="c", subcore_axis_name="s",
                                  num_cores=_NCORES, num_subcores=_NSUB)
    kern = pl.kernel(
        _dispatch_kernel,
        out_type=(
            jax.ShapeDtypeStruct((_NUM_EXPERTS, _CAPACITY, _EMBED), jnp.float32),
            jax.ShapeDtypeStruct((_NUM_EXPERTS, _CAPACITY), jnp.float32),
            jax.ShapeDtypeStruct((_NUM_EXPERTS, _CAPACITY), jnp.int32),
            jax.ShapeDtypeStruct((_L,), jnp.int32),
        ),
        mesh=mesh,
        compiler_params=pltpu.CompilerParams(needs_layout_passes=False),
        scratch_types=(
            pltpu.VMEM((_TOKENS,), jnp.int32),
            pltpu.VMEM((_IDXLEN,), jnp.int32),
            pltpu.VMEM((_TOKENS,), jnp.float32),
            pltpu.VMEM((_HALF,), jnp.float32),
            pltpu.VMEM((_HALF,), jnp.int32),
            pltpu.VMEM((_C, _EMBED), jnp.float32),
            pltpu.VMEM((_C, _EMBED), jnp.float32),
            pltpu.VMEM((_CZ, _EMBED), jnp.float32),
            pltpu.VMEM((_L,), jnp.int32),
            pltpu.VMEM((_NSUB, _L), jnp.int32),
            pltpu.VMEM_SHARED((_NSUB, _L), jnp.int32),
            pltpu.SemaphoreType.DMA,
            pltpu.SemaphoreType.DMA,
            pltpu.SemaphoreType.DMA,
        ),
    )
    xo, cw, ti, drop = kern(x, expert_ids, expert_weights)
    return xo, cw, ti, drop[0]


def kernel(x, expert_ids, expert_weights):
    return _dispatch(x, expert_ids, expert_weights)
